# baseline (speedup 1.0000x reference)
import jax
import jax.numpy as jnp
from jax.experimental import pallas as pl

D = 768
NUM_EXPERTS = 64
TOP_K = 2
ROUTER_DIM = 16
TOKENS = 32768

BLOCK = 2048


def _router_block(x_ref, wq_ref, keys_ref, scores_ref, idx_ref, probs_ref):
    q = jax.lax.dot_general(
        x_ref[...], wq_ref[...], (((1,), (1,)), ((), ())),
        preferred_element_type=jnp.float32,
    )
    scores = jax.lax.dot_general(
        q, keys_ref[...], (((1,), (1,)), ((), ())),
        preferred_element_type=jnp.float32,
    )
    scores_ref[...] = scores

    st = scores.T
    eidx = jax.lax.broadcasted_iota(jnp.int32, st.shape, 0)
    m1 = jnp.max(st, axis=0, keepdims=True)
    i1 = jnp.min(jnp.where(st == m1, eidx, NUM_EXPERTS),
                 axis=0, keepdims=True)
    masked = jnp.where(eidx == i1, -jnp.inf, st)
    m2 = jnp.max(masked, axis=0, keepdims=True)
    i2 = jnp.min(jnp.where(masked == m2, eidx, NUM_EXPERTS),
                 axis=0, keepdims=True)

    idx_ref[...] = jnp.concatenate([i1, i2], axis=0)
    e = jnp.exp(m2 - m1)
    denom = 1.0 + e
    probs_ref[...] = jnp.concatenate([1.0 / denom, e / denom], axis=0)


@jax.jit
def kernel(x, W_query, keys):
    scores, idx2, probs2 = pl.pallas_call(
        _router_block,
        grid=(TOKENS // BLOCK,),
        in_specs=[
            pl.BlockSpec((BLOCK, D), lambda i: (i, 0)),
            pl.BlockSpec((ROUTER_DIM, D), lambda i: (0, 0)),
            pl.BlockSpec((NUM_EXPERTS, ROUTER_DIM), lambda i: (0, 0)),
        ],
        out_specs=(
            pl.BlockSpec((BLOCK, NUM_EXPERTS), lambda i: (i, 0)),
            pl.BlockSpec((TOP_K, BLOCK), lambda i: (0, i)),
            pl.BlockSpec((TOP_K, BLOCK), lambda i: (0, i)),
        ),
        out_shape=(
            jax.ShapeDtypeStruct((TOKENS, NUM_EXPERTS), jnp.float32),
            jax.ShapeDtypeStruct((TOP_K, TOKENS), jnp.int32),
            jax.ShapeDtypeStruct((TOP_K, TOKENS), jnp.float32),
        ),
    )(x, W_query, keys)
    return idx2.T, probs2.T, scores


# speedup vs baseline: 1.0043x; 1.0043x over previous
import jax
import jax.numpy as jnp
from jax.experimental import pallas as pl
from jax.experimental.pallas import tpu as pltpu

D = 768
NUM_EXPERTS = 64
TOP_K = 2
ROUTER_DIM = 16
TOKENS = 32768

BLOCK = 8192


def _router_block(x_ref, wq_ref, keys_ref, scores_ref, idx_ref, probs_ref):
    q = jax.lax.dot_general(
        x_ref[...], wq_ref[...], (((1,), (1,)), ((), ())),
        preferred_element_type=jnp.float32,
    )
    scores = jax.lax.dot_general(
        q, keys_ref[...], (((1,), (1,)), ((), ())),
        preferred_element_type=jnp.float32,
    )
    scores_ref[...] = scores

    st = scores.T
    eidx = jax.lax.broadcasted_iota(jnp.int32, st.shape, 0)
    m1 = jnp.max(st, axis=0, keepdims=True)
    i1 = jnp.min(jnp.where(st == m1, eidx, NUM_EXPERTS),
                 axis=0, keepdims=True)
    masked = jnp.where(eidx == i1, -jnp.inf, st)
    m2 = jnp.max(masked, axis=0, keepdims=True)
    i2 = jnp.min(jnp.where(masked == m2, eidx, NUM_EXPERTS),
                 axis=0, keepdims=True)

    idx_ref[...] = jnp.concatenate([i1, i2], axis=0)
    e = jnp.exp(m2 - m1)
    denom = 1.0 + e
    probs_ref[...] = jnp.concatenate([1.0 / denom, e / denom], axis=0)


@jax.jit
def kernel(x, W_query, keys):
    scores, idx2, probs2 = pl.pallas_call(
        _router_block,
        grid=(TOKENS // BLOCK,),
        in_specs=[
            pl.BlockSpec((BLOCK, D), lambda i: (i, 0)),
            pl.BlockSpec((ROUTER_DIM, D), lambda i: (0, 0)),
            pl.BlockSpec((NUM_EXPERTS, ROUTER_DIM), lambda i: (0, 0)),
        ],
        out_specs=(
            pl.BlockSpec((BLOCK, NUM_EXPERTS), lambda i: (i, 0)),
            pl.BlockSpec((TOP_K, BLOCK), lambda i: (0, i)),
            pl.BlockSpec((TOP_K, BLOCK), lambda i: (0, i)),
        ),
        out_shape=(
            jax.ShapeDtypeStruct((TOKENS, NUM_EXPERTS), jnp.float32),
            jax.ShapeDtypeStruct((TOP_K, TOKENS), jnp.int32),
            jax.ShapeDtypeStruct((TOP_K, TOKENS), jnp.float32),
        ),
        compiler_params=pltpu.CompilerParams(
            vmem_limit_bytes=100 * 1024 * 1024,
        ),
    )(x, W_query, keys)
    return idx2.T, probs2.T, scores
